# baseline (speedup 1.0000x reference)
import functools

import jax
import jax.numpy as jnp
from jax import lax
from jax.experimental import pallas as pl
from jax.experimental.pallas import tpu as pltpu
from jax.experimental.pallas import tpu_sc as plsc

_B, _C, _H, _W = 8, 96, 224, 224
_N = 2000
_HW = _H * _W
_CHW = _C * _HW

_NUM_WORKERS = 32
_NPAD = 2048
_PTS = _NPAD // _NUM_WORKERS
_LANES = 16
_GCHUNK = 8

_RROWS = 512
_NQ = _B * _C * _H * _W // 128
_NU = _NQ // 7


def _repack_body(in_ref, out_ref):
    x = in_ref[...].reshape(_RROWS // 4, 4, _W)
    c = jnp.concatenate
    pieces = [
        x[:, 0, 0:128],
        c([x[:, 0, 128:224], x[:, 1, 0:32]], axis=1),
        x[:, 1, 32:160],
        c([x[:, 1, 160:224], x[:, 2, 0:64]], axis=1),
        x[:, 2, 64:192],
        c([x[:, 2, 192:224], x[:, 3, 0:96]], axis=1),
        x[:, 3, 96:224],
    ]
    out_ref[...] = jnp.stack(pieces, axis=0)


def _repack(table):
    rows = _B * _C * _H
    return pl.pallas_call(
        _repack_body,
        grid=(rows // _RROWS,),
        in_specs=[pl.BlockSpec((_RROWS, _W), lambda i: (i, 0))],
        out_specs=pl.BlockSpec((7, _RROWS // 4, 128), lambda i: (0, i, 0)),
        out_shape=jax.ShapeDtypeStruct((7, _NU, 128), jnp.float32),
    )(table)


def _sc_gather_kernel(inflat, ids_hbm, rows_hbm, cols_hbm, out_hbm,
                      ids_v, rows_v, cols_v, idx_v, g_v, sem):
    nc = lax.axis_size("c")
    wid = lax.axis_index("s") * nc + lax.axis_index("c")
    base = wid * _PTS

    pltpu.sync_copy(ids_hbm.at[pl.ds(base, _PTS)], ids_v)
    pltpu.sync_copy(rows_hbm.at[pl.ds(base, _PTS)], rows_v)
    pltpu.sync_copy(cols_hbm.at[pl.ds(base, _PTS)], cols_v)

    pbases = []
    for gr in range(_PTS // _LANES):
        sl = pl.ds(gr * _LANES, _LANES)
        bv = ids_v[sl] * _CHW + rows_v[sl] * _W + cols_v[sl]
        lane0 = jnp.bitwise_and(bv, 127)
        q0 = lax.shift_right_logical(bv, 7)
        m0 = ((q0.astype(jnp.float32) + 0.5) * (1.0 / 7.0)).astype(jnp.int32)
        t0 = q0 - 7 * m0
        pbases.append(t0 * (_NU * 128) + m0 * 128 + lane0)

    _CSTEP = _HW // 7
    def build(c, carry):
        coff = c * _CSTEP
        for gr in range(_PTS // _LANES):
            idx_v[pl.ds(c * _PTS + gr * _LANES, _LANES)] = pbases[gr] + coff
        return carry

    lax.fori_loop(0, _C, build, 0)

    def gather_chunk(i, carry):
        cb = i * _GCHUNK
        descs = []
        for j in range(_GCHUNK):
            c = cb + j
            descs.append(pltpu.async_copy(
                inflat.at[idx_v.at[pl.ds(c * _PTS, _PTS)]], g_v.at[c], sem))
        for d in descs:
            d.wait()
        return carry

    lax.fori_loop(0, _C // _GCHUNK, gather_chunk, 0)

    pltpu.sync_copy(g_v, out_hbm.at[wid])


@functools.cache
def _sc_gather():
    return pl.kernel(
        _sc_gather_kernel,
        out_type=jax.ShapeDtypeStruct((_NUM_WORKERS, _C, _PTS), jnp.float32),
        mesh=plsc.VectorSubcoreMesh(
            core_axis_name="c", subcore_axis_name="s",
            num_cores=2, num_subcores=16,
        ),
        scratch_types=[
            pltpu.VMEM((_PTS,), jnp.int32),
            pltpu.VMEM((_PTS,), jnp.int32),
            pltpu.VMEM((_PTS,), jnp.int32),
            pltpu.VMEM((_C * _PTS,), jnp.int32),
            pltpu.VMEM((_C, _PTS), jnp.float32),
            pltpu.SemaphoreType.DMA,
        ],
    )


def _bcast_body(g_ref, out_ref):
    g = g_ref[0]
    out_ref[...] = jnp.broadcast_to(g.T[:, :, None], out_ref.shape)


def _tc_broadcast(g):
    return pl.pallas_call(
        _bcast_body,
        grid=(_NUM_WORKERS,),
        in_specs=[pl.BlockSpec((1, _C, _PTS), lambda i: (i, 0, 0))],
        out_specs=pl.BlockSpec((_PTS, _C, _W), lambda i: (i, 0, 0)),
        out_shape=jax.ShapeDtypeStruct((_N, _C, _W), jnp.float32),
    )(g)


def kernel(input, image_ids, centers):
    pad = _NPAD - _N
    ids = jnp.pad(image_ids.astype(jnp.int32), (0, pad))
    rows = jnp.pad(centers[:, 0].astype(jnp.int32), (0, pad))
    cols = jnp.pad(centers[:, 1].astype(jnp.int32), (0, pad))
    table = input.reshape(_B * _C * _H, _W)
    inflat = _repack(table).reshape(-1)
    g = _sc_gather()(inflat, ids, rows, cols)
    return _tc_broadcast(g)


# speedup vs baseline: 1.1920x; 1.1920x over previous
import functools

import jax
import jax.numpy as jnp
from jax import lax
from jax.experimental import pallas as pl
from jax.experimental.pallas import tpu as pltpu
from jax.experimental.pallas import tpu_sc as plsc

_B, _C, _H, _W = 8, 96, 224, 224
_N = 2000
_HW = _H * _W
_CHW = _C * _HW

_NUM_WORKERS = 32
_NPAD = 2048
_PTS = _NPAD // _NUM_WORKERS
_LANES = 16
_GCHUNK = 8

_RROWS = 512
_ROWS = _B * _C * _H
_PLANE = _ROWS * 128


def _repack_body(in_ref, out_ref):
    x = in_ref[...]
    out_ref[0] = x[:, 0:128]
    out_ref[1] = x[:, 96:224]


def _repack(table):
    return pl.pallas_call(
        _repack_body,
        grid=(_ROWS // _RROWS,),
        in_specs=[pl.BlockSpec((_RROWS, _W), lambda i: (i, 0))],
        out_specs=pl.BlockSpec((2, _RROWS, 128), lambda i: (0, i, 0)),
        out_shape=jax.ShapeDtypeStruct((2, _ROWS, 128), jnp.float32),
    )(table)


def _sc_gather_kernel(inflat, ids_hbm, rows_hbm, cols_hbm, out_hbm,
                      ids_v, rows_v, cols_v, idx_v, g_v, sem):
    nc = lax.axis_size("c")
    wid = lax.axis_index("s") * nc + lax.axis_index("c")
    base = wid * _PTS

    pltpu.sync_copy(ids_hbm.at[pl.ds(base, _PTS)], ids_v)
    pltpu.sync_copy(rows_hbm.at[pl.ds(base, _PTS)], rows_v)
    pltpu.sync_copy(cols_hbm.at[pl.ds(base, _PTS)], cols_v)

    pbases = []
    for gr in range(_PTS // _LANES):
        sl = pl.ds(gr * _LANES, _LANES)
        col = cols_v[sl]
        sel = lax.shift_right_logical(col, 7)
        rbase = ids_v[sl] * (_C * _H) + rows_v[sl]
        pbases.append(sel * _PLANE + rbase * 128 + col - 96 * sel)

    def build(c, carry):
        coff = c * (_H * 128)
        for gr in range(_PTS // _LANES):
            idx_v[pl.ds(c * _PTS + gr * _LANES, _LANES)] = pbases[gr] + coff
        return carry

    lax.fori_loop(0, _C, build, 0)

    def gather_chunk(i, carry):
        cb = i * _GCHUNK
        descs = []
        for j in range(_GCHUNK):
            c = cb + j
            descs.append(pltpu.async_copy(
                inflat.at[idx_v.at[pl.ds(c * _PTS, _PTS)]], g_v.at[c], sem))
        for d in descs:
            d.wait()
        return carry

    lax.fori_loop(0, _C // _GCHUNK, gather_chunk, 0)

    pltpu.sync_copy(g_v, out_hbm.at[wid])


@functools.cache
def _sc_gather():
    return pl.kernel(
        _sc_gather_kernel,
        out_type=jax.ShapeDtypeStruct((_NUM_WORKERS, _C, _PTS), jnp.float32),
        mesh=plsc.VectorSubcoreMesh(
            core_axis_name="c", subcore_axis_name="s",
            num_cores=2, num_subcores=16,
        ),
        scratch_types=[
            pltpu.VMEM((_PTS,), jnp.int32),
            pltpu.VMEM((_PTS,), jnp.int32),
            pltpu.VMEM((_PTS,), jnp.int32),
            pltpu.VMEM((_C * _PTS,), jnp.int32),
            pltpu.VMEM((_C, _PTS), jnp.float32),
            pltpu.SemaphoreType.DMA,
        ],
    )


def _bcast_body(g_ref, out_ref):
    g = g_ref[0]
    out_ref[...] = jnp.broadcast_to(g.T[:, :, None], out_ref.shape)


def _tc_broadcast(g):
    return pl.pallas_call(
        _bcast_body,
        grid=(_NUM_WORKERS,),
        in_specs=[pl.BlockSpec((1, _C, _PTS), lambda i: (i, 0, 0))],
        out_specs=pl.BlockSpec((_PTS, _C, _W), lambda i: (i, 0, 0)),
        out_shape=jax.ShapeDtypeStruct((_N, _C, _W), jnp.float32),
    )(g)


def kernel(input, image_ids, centers):
    pad = _NPAD - _N
    ids = jnp.pad(image_ids.astype(jnp.int32), (0, pad))
    rows = jnp.pad(centers[:, 0].astype(jnp.int32), (0, pad))
    cols = jnp.pad(centers[:, 1].astype(jnp.int32), (0, pad))
    table = input.reshape(_B * _C * _H, _W)
    inflat = _repack(table).reshape(-1)
    g = _sc_gather()(inflat, ids, rows, cols)
    return _tc_broadcast(g)


# speedup vs baseline: 1.3650x; 1.1451x over previous
import functools

import jax
import jax.numpy as jnp
from jax import lax
from jax.experimental import pallas as pl
from jax.experimental.pallas import tpu as pltpu
from jax.experimental.pallas import tpu_sc as plsc

_B, _C, _H, _W = 8, 96, 224, 224
_N = 2000

_NUM_WORKERS = 32
_NPAD = 2048
_PTS = _NPAD // _NUM_WORKERS
_LANES = 16
_GCHUNK = 8

_RROWS = 1024
_ROWS = _B * _C * _H
_PLANE = _ROWS * 128


def _repack_body(in_ref, a_ref, b_ref):
    x = in_ref[...]
    a_ref[...] = x[:, 0:128].reshape(a_ref.shape)
    b_ref[...] = x[:, 96:224].reshape(b_ref.shape)


def _repack(table):
    blk = _RROWS * 128
    return pl.pallas_call(
        _repack_body,
        grid=(_ROWS // _RROWS,),
        in_specs=[pl.BlockSpec((_RROWS, _W), lambda i: (i, 0))],
        out_specs=[pl.BlockSpec((blk,), lambda i: (i,)),
                   pl.BlockSpec((blk,), lambda i: (i,))],
        out_shape=[jax.ShapeDtypeStruct((_PLANE,), jnp.float32),
                   jax.ShapeDtypeStruct((_PLANE,), jnp.float32)],
    )(table)


def _sc_gather_kernel(pa, pb, ids_hbm, rows_hbm, cols_hbm, out_hbm,
                      ids_v, rows_v, cols_v, idxa_v, idxb_v,
                      ga_v, gb_v, g_v, sem):
    nc = lax.axis_size("c")
    wid = lax.axis_index("s") * nc + lax.axis_index("c")
    base = wid * _PTS

    pltpu.sync_copy(ids_hbm.at[pl.ds(base, _PTS)], ids_v)
    pltpu.sync_copy(rows_hbm.at[pl.ds(base, _PTS)], rows_v)
    pltpu.sync_copy(cols_hbm.at[pl.ds(base, _PTS)], cols_v)

    pbase_a, pbase_b, self_f = [], [], []
    for gr in range(_PTS // _LANES):
        sl = pl.ds(gr * _LANES, _LANES)
        col = cols_v[sl]
        rbase = ids_v[sl] * (_C * _H) + rows_v[sl]
        sel = lax.shift_right_logical(col, 7)
        pbase_a.append(rbase * 128 + jnp.minimum(col, 127))
        pbase_b.append(rbase * 128 + jnp.maximum(col, 96) - 96)
        self_f.append(sel.astype(jnp.float32))

    def build(c, carry):
        coff = c * (_H * 128)
        for gr in range(_PTS // _LANES):
            sl = pl.ds(c * _PTS + gr * _LANES, _LANES)
            idxa_v[sl] = pbase_a[gr] + coff
            idxb_v[sl] = pbase_b[gr] + coff
        return carry

    lax.fori_loop(0, _C, build, 0)

    def gather_chunk(i, carry):
        cb = i * _GCHUNK
        descs = []
        for j in range(_GCHUNK):
            c = cb + j
            descs.append(pltpu.async_copy(
                pa.at[idxa_v.at[pl.ds(c * _PTS, _PTS)]], ga_v.at[c], sem))
            descs.append(pltpu.async_copy(
                pb.at[idxb_v.at[pl.ds(c * _PTS, _PTS)]], gb_v.at[c], sem))
        for d in descs:
            d.wait()
        return carry

    lax.fori_loop(0, _C // _GCHUNK, gather_chunk, 0)

    def select(c, carry):
        for gr in range(_PTS // _LANES):
            sl = pl.ds(gr * _LANES, _LANES)
            a = ga_v[c, sl]
            b = gb_v[c, sl]
            g_v[c, sl] = a + (b - a) * self_f[gr]
        return carry

    lax.fori_loop(0, _C, select, 0)

    pltpu.sync_copy(g_v, out_hbm.at[wid])


@functools.cache
def _sc_gather():
    return pl.kernel(
        _sc_gather_kernel,
        out_type=jax.ShapeDtypeStruct((_NUM_WORKERS, _C, _PTS), jnp.float32),
        mesh=plsc.VectorSubcoreMesh(
            core_axis_name="c", subcore_axis_name="s",
            num_cores=2, num_subcores=16,
        ),
        scratch_types=[
            pltpu.VMEM((_PTS,), jnp.int32),
            pltpu.VMEM((_PTS,), jnp.int32),
            pltpu.VMEM((_PTS,), jnp.int32),
            pltpu.VMEM((_C * _PTS,), jnp.int32),
            pltpu.VMEM((_C * _PTS,), jnp.int32),
            pltpu.VMEM((_C, _PTS), jnp.float32),
            pltpu.VMEM((_C, _PTS), jnp.float32),
            pltpu.VMEM((_C, _PTS), jnp.float32),
            pltpu.SemaphoreType.DMA,
        ],
    )


def _bcast_body(g_ref, out_ref):
    g = g_ref[0]
    out_ref[...] = jnp.broadcast_to(g.T[:, :, None], out_ref.shape)


def _tc_broadcast(g):
    return pl.pallas_call(
        _bcast_body,
        grid=(_NUM_WORKERS,),
        in_specs=[pl.BlockSpec((1, _C, _PTS), lambda i: (i, 0, 0))],
        out_specs=pl.BlockSpec((_PTS, _C, _W), lambda i: (i, 0, 0)),
        out_shape=jax.ShapeDtypeStruct((_N, _C, _W), jnp.float32),
    )(g)


def kernel(input, image_ids, centers):
    pad = _NPAD - _N
    ids = jnp.pad(image_ids.astype(jnp.int32), (0, pad))
    rows = jnp.pad(centers[:, 0].astype(jnp.int32), (0, pad))
    cols = jnp.pad(centers[:, 1].astype(jnp.int32), (0, pad))
    table = input.reshape(_ROWS, _W)
    pa, pb = _repack(table)
    g = _sc_gather()(pa, pb, ids, rows, cols)
    return _tc_broadcast(g)


# speedup vs baseline: 1.5215x; 1.1147x over previous
import functools

import jax
import jax.numpy as jnp
from jax import lax
from jax.experimental import pallas as pl
from jax.experimental.pallas import tpu as pltpu
from jax.experimental.pallas import tpu_sc as plsc

_B, _C, _H, _W = 8, 96, 224, 224
_N = 2000

_NUM_WORKERS = 32
_NPAD = 2048
_PTS = _NPAD // _NUM_WORKERS
_LANES = 16
_GCHUNK = 8

_RROWS = 1024
_ROWS = _B * _C * _H
_PLANE = _ROWS * 128


_CPB = 8


def _repack_body(in_ref, a_ref, b_ref):
    x = in_ref[0]
    a_ref[...] = x[:, :, 0:128].reshape(a_ref.shape)
    b_ref[...] = x[:, :, 96:224].reshape(b_ref.shape)


def _repack(input):
    blk = _CPB * _H * 128
    nc = _C // _CPB
    return pl.pallas_call(
        _repack_body,
        grid=(_B * nc,),
        in_specs=[pl.BlockSpec((1, _CPB, _H, _W),
                               lambda i: (i // nc, i % nc, 0, 0))],
        out_specs=[pl.BlockSpec((blk,), lambda i: (i,)),
                   pl.BlockSpec((blk,), lambda i: (i,))],
        out_shape=[jax.ShapeDtypeStruct((_PLANE,), jnp.float32),
                   jax.ShapeDtypeStruct((_PLANE,), jnp.float32)],
    )(input)


def _sc_gather_kernel(pa, pb, ids_hbm, rows_hbm, cols_hbm, out_hbm,
                      ids_v, rows_v, cols_v, idxa_v, idxb_v,
                      ga_v, gb_v, g_v, sem):
    nc = lax.axis_size("c")
    wid = lax.axis_index("s") * nc + lax.axis_index("c")
    base = wid * _PTS

    pltpu.sync_copy(ids_hbm.at[pl.ds(base, _PTS)], ids_v)
    pltpu.sync_copy(rows_hbm.at[pl.ds(base, _PTS)], rows_v)
    pltpu.sync_copy(cols_hbm.at[pl.ds(base, _PTS)], cols_v)

    pbase_a, pbase_b, self_f = [], [], []
    for gr in range(_PTS // _LANES):
        sl = pl.ds(gr * _LANES, _LANES)
        col = cols_v[sl]
        rbase = ids_v[sl] * (_C * _H) + rows_v[sl]
        sel = lax.shift_right_logical(col, 7)
        pbase_a.append(rbase * 128 + jnp.minimum(col, 127))
        pbase_b.append(rbase * 128 + jnp.maximum(col, 96) - 96)
        self_f.append(sel.astype(jnp.float32))

    def build(c, carry):
        coff = c * (_H * 128)
        for gr in range(_PTS // _LANES):
            sl = pl.ds(c * _PTS + gr * _LANES, _LANES)
            idxa_v[sl] = pbase_a[gr] + coff
            idxb_v[sl] = pbase_b[gr] + coff
        return carry

    lax.fori_loop(0, _C, build, 0)

    def gather_chunk(i, carry):
        cb = i * _GCHUNK
        descs = []
        for j in range(_GCHUNK):
            c = cb + j
            descs.append(pltpu.async_copy(
                pa.at[idxa_v.at[pl.ds(c * _PTS, _PTS)]], ga_v.at[c], sem))
            descs.append(pltpu.async_copy(
                pb.at[idxb_v.at[pl.ds(c * _PTS, _PTS)]], gb_v.at[c], sem))
        for d in descs:
            d.wait()
        return carry

    lax.fori_loop(0, _C // _GCHUNK, gather_chunk, 0)

    def select(c, carry):
        for gr in range(_PTS // _LANES):
            sl = pl.ds(gr * _LANES, _LANES)
            a = ga_v[c, sl]
            b = gb_v[c, sl]
            g_v[c, sl] = a + (b - a) * self_f[gr]
        return carry

    lax.fori_loop(0, _C, select, 0)

    pltpu.sync_copy(g_v, out_hbm.at[wid])


@functools.cache
def _sc_gather():
    return pl.kernel(
        _sc_gather_kernel,
        out_type=jax.ShapeDtypeStruct((_NUM_WORKERS, _C, _PTS), jnp.float32),
        mesh=plsc.VectorSubcoreMesh(
            core_axis_name="c", subcore_axis_name="s",
            num_cores=2, num_subcores=16,
        ),
        scratch_types=[
            pltpu.VMEM((_PTS,), jnp.int32),
            pltpu.VMEM((_PTS,), jnp.int32),
            pltpu.VMEM((_PTS,), jnp.int32),
            pltpu.VMEM((_C * _PTS,), jnp.int32),
            pltpu.VMEM((_C * _PTS,), jnp.int32),
            pltpu.VMEM((_C, _PTS), jnp.float32),
            pltpu.VMEM((_C, _PTS), jnp.float32),
            pltpu.VMEM((_C, _PTS), jnp.float32),
            pltpu.SemaphoreType.DMA,
        ],
    )


def _bcast_body(g_ref, out_ref):
    g = g_ref[0]
    out_ref[...] = jnp.broadcast_to(g.T[:, :, None], out_ref.shape)


def _tc_broadcast(g):
    return pl.pallas_call(
        _bcast_body,
        grid=(_NUM_WORKERS,),
        in_specs=[pl.BlockSpec((1, _C, _PTS), lambda i: (i, 0, 0))],
        out_specs=pl.BlockSpec((_PTS, _C, _W), lambda i: (i, 0, 0)),
        out_shape=jax.ShapeDtypeStruct((_N, _C, _W), jnp.float32),
    )(g)


def kernel(input, image_ids, centers):
    pad = _NPAD - _N
    ids = jnp.pad(image_ids.astype(jnp.int32), (0, pad))
    rows = jnp.pad(centers[:, 0].astype(jnp.int32), (0, pad))
    cols = jnp.pad(centers[:, 1].astype(jnp.int32), (0, pad))
    pa, pb = _repack(input)
    g = _sc_gather()(pa, pb, ids, rows, cols)
    return _tc_broadcast(g)


# speedup vs baseline: 2.6190x; 1.7213x over previous
import functools

import jax
import jax.numpy as jnp
from jax import lax
from jax.experimental import pallas as pl
from jax.experimental.pallas import tpu as pltpu
from jax.experimental.pallas import tpu_sc as plsc

_B, _C, _H, _W = 8, 96, 224, 224
_N = 2000

_NUM_WORKERS = 32
_NPAD = 2048
_PTS = _NPAD // _NUM_WORKERS
_LANES = 16
_GCHUNK = 8

_RROWS = 1024
_ROWS = _B * _C * _H
_PLANE = _ROWS * 128


_CPB = 8


def _repack_body(in_ref, a_ref, b_ref):
    x = in_ref[0]
    a_ref[...] = x[:, :, 0:128].reshape(a_ref.shape)
    b_ref[...] = x[:, :, 96:224].reshape(b_ref.shape)


def _repack(input):
    blk = _CPB * _H * 128
    nc = _C // _CPB
    return pl.pallas_call(
        _repack_body,
        grid=(_B * nc,),
        in_specs=[pl.BlockSpec((1, _CPB, _H, _W),
                               lambda i: (i // nc, i % nc, 0, 0))],
        out_specs=[pl.BlockSpec((blk,), lambda i: (i,)),
                   pl.BlockSpec((blk,), lambda i: (i,))],
        out_shape=[jax.ShapeDtypeStruct((_PLANE,), jnp.float32),
                   jax.ShapeDtypeStruct((_PLANE,), jnp.float32)],
    )(input)


def _sc_gather_kernel(pa, pb, ids_hbm, rows_hbm, cols_hbm, out_hbm,
                      ids_v, rows_v, cols_v, idxa_v, idxb_v,
                      ga_v, gb_v, g_v, sem):
    nc = lax.axis_size("c")
    wid = lax.axis_index("s") * nc + lax.axis_index("c")
    base = wid * _PTS

    pltpu.sync_copy(ids_hbm.at[pl.ds(base, _PTS)], ids_v)
    pltpu.sync_copy(rows_hbm.at[pl.ds(base, _PTS)], rows_v)
    pltpu.sync_copy(cols_hbm.at[pl.ds(base, _PTS)], cols_v)

    pbase_a, pbase_b, self_f = [], [], []
    for gr in range(_PTS // _LANES):
        sl = pl.ds(gr * _LANES, _LANES)
        col = cols_v[sl]
        rbase = ids_v[sl] * (_C * _H) + rows_v[sl]
        sel = lax.shift_right_logical(col, 7)
        pbase_a.append(rbase * 128 + jnp.minimum(col, 127))
        pbase_b.append(rbase * 128 + jnp.maximum(col, 96) - 96)
        self_f.append(sel.astype(jnp.float32))

    def build(c, carry):
        coff = c * (_H * 128)
        for gr in range(_PTS // _LANES):
            sl = pl.ds(c * _PTS + gr * _LANES, _LANES)
            idxa_v[sl] = pbase_a[gr] + coff
            idxb_v[sl] = pbase_b[gr] + coff
        return carry

    lax.fori_loop(0, _C, build, 0)

    def gather_chunk(i, carry):
        cb = i * _GCHUNK
        descs = []
        for j in range(_GCHUNK):
            c = cb + j
            descs.append(pltpu.async_copy(
                pa.at[idxa_v.at[pl.ds(c * _PTS, _PTS)]], ga_v.at[c], sem))
            descs.append(pltpu.async_copy(
                pb.at[idxb_v.at[pl.ds(c * _PTS, _PTS)]], gb_v.at[c], sem))
        for d in descs:
            d.wait()
        return carry

    lax.fori_loop(0, _C // _GCHUNK, gather_chunk, 0)

    def select(c, carry):
        for gr in range(_PTS // _LANES):
            sl = pl.ds(gr * _LANES, _LANES)
            a = ga_v[c, sl]
            b = gb_v[c, sl]
            g_v[c, sl] = a + (b - a) * self_f[gr]
        return carry

    lax.fori_loop(0, _C, select, 0)

    pltpu.sync_copy(g_v, out_hbm.at[wid])


@functools.cache
def _sc_gather():
    return pl.kernel(
        _sc_gather_kernel,
        out_type=jax.ShapeDtypeStruct((_NUM_WORKERS, _C, _PTS), jnp.float32),
        mesh=plsc.VectorSubcoreMesh(
            core_axis_name="c", subcore_axis_name="s",
            num_cores=2, num_subcores=16,
        ),
        scratch_types=[
            pltpu.VMEM((_PTS,), jnp.int32),
            pltpu.VMEM((_PTS,), jnp.int32),
            pltpu.VMEM((_PTS,), jnp.int32),
            pltpu.VMEM((_C * _PTS,), jnp.int32),
            pltpu.VMEM((_C * _PTS,), jnp.int32),
            pltpu.VMEM((_C, _PTS), jnp.float32),
            pltpu.VMEM((_C, _PTS), jnp.float32),
            pltpu.VMEM((_C, _PTS), jnp.float32),
            pltpu.SemaphoreType.DMA,
        ],
    )


_WCHUNK = 8


def _bcast_body(g_ref, out_ref):
    g = g_ref[...]
    m = jnp.concatenate([g[i] for i in range(_NUM_WORKERS)], axis=1)
    out_ref[...] = jnp.broadcast_to(m[:, None, :_N], out_ref.shape)


def _tc_broadcast(g):
    return pl.pallas_call(
        _bcast_body,
        grid=(_W // _WCHUNK,),
        in_specs=[pl.BlockSpec((_NUM_WORKERS, _C, _PTS), lambda i: (0, 0, 0))],
        out_specs=pl.BlockSpec((_C, _WCHUNK, _N), lambda i: (0, i, 0)),
        out_shape=jax.ShapeDtypeStruct((_C, _W, _N), jnp.float32),
    )(g)


def kernel(input, image_ids, centers):
    pad = _NPAD - _N
    ids = jnp.pad(image_ids.astype(jnp.int32), (0, pad))
    rows = jnp.pad(centers[:, 0].astype(jnp.int32), (0, pad))
    cols = jnp.pad(centers[:, 1].astype(jnp.int32), (0, pad))
    pa, pb = _repack(input)
    g = _sc_gather()(pa, pb, ids, rows, cols)
    return jnp.transpose(_tc_broadcast(g), (2, 0, 1))
